# baseline (speedup 1.0000x reference)
import functools

import jax
import jax.numpy as jnp
from jax.experimental import pallas as pl
from jax.experimental.pallas import tpu as pltpu

_BR = 256
_L = 128


def _cumsum_kernel(t_ref, x_ref, o_ref, *, br, g, l):
    xb = x_ref[...]
    x2 = xb.reshape(br * g, l)
    s2 = jnp.dot(x2, t_ref[...], preferred_element_type=jnp.float32)
    s3 = s2.reshape(br, g, l)

    tot = jnp.sum(xb, axis=2)
    g_idx = jax.lax.broadcasted_iota(jnp.int32, (br, g), 1)
    acc = tot
    d = 1
    while d < g:
        rolled = pltpu.roll(acc, d, 1)
        acc = acc + jnp.where(g_idx >= d, rolled, 0.0)
        d *= 2
    excl = acc - tot

    o_ref[...] = s3 + excl[:, :, None]


@jax.jit
def kernel(x):
    m, n = x.shape
    g = n // _L
    xr = x.reshape(m, g, _L)
    tri = jnp.triu(jnp.ones((_L, _L), dtype=jnp.float32))
    out = pl.pallas_call(
        functools.partial(_cumsum_kernel, br=_BR, g=g, l=_L),
        grid=(m // _BR,),
        in_specs=[
            pl.BlockSpec((_L, _L), lambda i: (0, 0)),
            pl.BlockSpec((_BR, g, _L), lambda i: (i, 0, 0)),
        ],
        out_specs=pl.BlockSpec((_BR, g, _L), lambda i: (i, 0, 0)),
        out_shape=jax.ShapeDtypeStruct((m, g, _L), x.dtype),
        compiler_params=pltpu.CompilerParams(
            dimension_semantics=("parallel",)
        ),
    )(tri, xr)
    return out.reshape(m, n)


# speedup vs baseline: 8.0432x; 8.0432x over previous
import functools

import jax
import jax.numpy as jnp
from jax.experimental import pallas as pl
from jax.experimental.pallas import tpu as pltpu

_BR = 256
_L = 128


def _cumsum_kernel(w_ref, x_ref, o_ref, *, br, n, l):
    w = w_ref[...]
    carry = jnp.zeros((br, l), dtype=jnp.float32)
    for c in range(n // l):
        xc = x_ref[:, c * l : (c + 1) * l]
        y = jnp.dot(xc, w, preferred_element_type=jnp.float32)
        o_ref[:, c * l : (c + 1) * l] = y[:, :l] + carry
        carry = carry + y[:, l:]


@jax.jit
def kernel(x):
    m, n = x.shape
    tri = jnp.triu(jnp.ones((_L, _L), dtype=jnp.float32))
    w = jnp.concatenate([tri, jnp.ones((_L, _L), dtype=jnp.float32)], axis=1)
    return pl.pallas_call(
        functools.partial(_cumsum_kernel, br=_BR, n=n, l=_L),
        grid=(m // _BR,),
        in_specs=[
            pl.BlockSpec((_L, 2 * _L), lambda i: (0, 0)),
            pl.BlockSpec((_BR, n), lambda i: (i, 0)),
        ],
        out_specs=pl.BlockSpec((_BR, n), lambda i: (i, 0)),
        out_shape=jax.ShapeDtypeStruct((m, n), x.dtype),
        compiler_params=pltpu.CompilerParams(
            dimension_semantics=("parallel",)
        ),
    )(w, x)
